# Initial kernel scaffold; baseline (speedup 1.0000x reference)
import jax
import jax.numpy as jnp
from jax.experimental import pallas as pl


def kernel(params, input_ids, decoder_input_ids):
    raise NotImplementedError("write your pallas kernel here")


# speedup vs baseline: 1.2856x; 1.2856x over previous
import functools

import jax
import jax.numpy as jnp
from jax import lax
from jax.experimental import pallas as pl
from jax.experimental.pallas import tpu as pltpu
from jax.experimental.pallas import tpu_sc as plsc

D = 768
VOCAB = 32128
KH = 32
NH = 12
DH = 64
LD = 64
T = 2048
QB = 1024
KB = 1024
VB = 512
NEG = -1e9
F32 = jnp.float32


_SC_CORES = 2
_SC_SUBCORES = 16
_SC_WORKERS = _SC_CORES * _SC_SUBCORES


def _embed(table, ids):
    b = ids.shape[0]
    bpw = b // _SC_WORKERS
    mesh = plsc.VectorSubcoreMesh(core_axis_name="c", subcore_axis_name="s")

    @functools.partial(
        pl.kernel,
        mesh=mesh,
        out_type=jax.ShapeDtypeStruct((b, D), F32),
        scratch_types=[
            pltpu.VMEM((bpw,), jnp.int32),
            pltpu.VMEM((bpw, D), F32),
            pltpu.SemaphoreType.DMA,
        ],
    )
    def gather_k(table_hbm, idx_hbm, out_hbm, idx_v, rows_v, sem):
        wid = lax.axis_index("s") * _SC_CORES + lax.axis_index("c")
        base = wid * bpw
        pltpu.sync_copy(idx_hbm.at[pl.ds(base, bpw)], idx_v)
        pltpu.async_copy(table_hbm.at[idx_v], rows_v, sem).wait()
        pltpu.sync_copy(rows_v, out_hbm.at[pl.ds(base, bpw)])

    return gather_k(table, ids)


_INV_D = 1.0 / D


def _ln2(x):
    mu = jnp.sum(x, axis=-1, keepdims=True) * _INV_D
    var = jnp.sum(jnp.square(x - mu), axis=-1, keepdims=True) * _INV_D
    return (x - mu) / jnp.sqrt(var + 1e-5)


def _softmax(l):
    m = jnp.max(l, axis=-1, keepdims=True)
    e = jnp.exp(l - m)
    return e / jnp.sum(e, axis=-1, keepdims=True)


def _dot(a, b):
    return jnp.dot(a, b, preferred_element_type=F32)


def _dot_t(a, b):
    return lax.dot_general(a, b, (((1,), (1,)), ((), ())),
                           preferred_element_type=F32)


def _topk_smem(h_ref, r_ref, idx_s, val_s):
    s0 = _dot_t(r_ref[...], h_ref[...])
    col = lax.broadcasted_iota(jnp.int32, s0.shape, 1)
    n = s0.shape[1]

    def body(j, s):
        m = jnp.max(s)
        idx = jnp.min(jnp.where(s == m, col, n))
        idx_s[j] = idx
        val_s[j] = m
        return jnp.where(col == idx, -jnp.inf, s)

    lax.fori_loop(0, KH, body, s0)


def _col_f32(s_ref, n):
    return jnp.concatenate(
        [jnp.full((1, 1), s_ref[j], F32) for j in range(n)], axis=0)


def _col_i32(s_ref, n):
    return jnp.concatenate(
        [jnp.full((1, 1), s_ref[j], jnp.int32) for j in range(n)], axis=0)


def _row_i32(s_ref, n):
    return jnp.concatenate(
        [jnp.full((1, 1), s_ref[j], jnp.int32) for j in range(n)], axis=1)


def _ln_body(x_ref, o_ref):
    o_ref[...] = _ln2(x_ref[...])


def _ln(x):
    return pl.pallas_call(
        _ln_body,
        grid=(8,),
        in_specs=[pl.BlockSpec((T // 8, D), lambda i: (i, 0))],
        out_specs=pl.BlockSpec((T // 8, D), lambda i: (i, 0)),
        out_shape=jax.ShapeDtypeStruct((T, D), F32),
    )(x)


def _light_body(causal, x_ref, hq_ref, hkv_ref, wq_ref, wk_ref, wv_ref,
                wo_ref, o_ref):
    i = pl.program_id(0)
    q = _dot(hq_ref[...], wq_ref[...]).astype(jnp.bfloat16)
    k = _dot(hkv_ref[...], wk_ref[...]).astype(jnp.bfloat16)
    v = _dot(hkv_ref[...], wv_ref[...])
    m = jnp.full((QB, 1), -jnp.inf, F32)
    s = jnp.zeros((QB, 1), F32)
    acc = jnp.zeros((QB, LD), F32)
    for kb in range(T // KB):
        kblk = k[kb * KB:(kb + 1) * KB, :]
        l = lax.dot_general(q, kblk, (((1,), (1,)), ((), ())),
                            preferred_element_type=F32) * 0.125
        if causal:
            rows = i * QB + lax.broadcasted_iota(jnp.int32, (QB, KB), 0)
            cols = kb * KB + lax.broadcasted_iota(jnp.int32, (QB, KB), 1)
            l = jnp.where(cols <= rows, l, NEG)
        mb = jnp.max(l, axis=1, keepdims=True)
        mnew = jnp.maximum(m, mb)
        delta = jnp.where(m == mnew, 0.0, m - mnew)
        e = jnp.exp(l - mnew)
        bs = jnp.sum(e, axis=1, keepdims=True)
        t = jnp.exp(delta) * s
        snew = t + bs
        raw = _dot(e, v[kb * KB:(kb + 1) * KB, :]) + t * acc
        acc = raw * (1.0 / snew)
        m, s = mnew, snew
    o_ref[...] = x_ref[...] + _dot(acc, wo_ref[...])


def _light(x, hq, hkv, wq, wk, wv, wo, causal):
    return pl.pallas_call(
        functools.partial(_light_body, causal),
        grid=(T // QB,),
        in_specs=[
            pl.BlockSpec((QB, D), lambda i: (i, 0)),
            pl.BlockSpec((QB, D), lambda i: (i, 0)),
            pl.BlockSpec((T, D), lambda i: (0, 0)),
            pl.BlockSpec((D, LD), lambda i: (0, 0)),
            pl.BlockSpec((D, LD), lambda i: (0, 0)),
            pl.BlockSpec((D, LD), lambda i: (0, 0)),
            pl.BlockSpec((LD, D), lambda i: (0, 0)),
        ],
        out_specs=pl.BlockSpec((QB, D), lambda i: (i, 0)),
        out_shape=jax.ShapeDtypeStruct((T, D), F32),
    )(x, hq, hkv, wq, wk, wv, wo)


def _heavy_body(causal, x_ref, hq_ref, hkv_ref, rq_ref, rkv_ref, wq_ref,
                wk_ref, wv_ref, wo_ref, o_ref, qi_s, qv_s, ki_s, kv_s,
                qs_ref, ks_ref):
    _topk_smem(hq_ref, rq_ref, qi_s, qv_s)
    _topk_smem(hkv_ref, rkv_ref, ki_s, kv_s)
    for j in range(KH):
        qs_ref[pl.ds(j, 1), :] = hq_ref[pl.ds(qi_s[j], 1), :]
        ks_ref[pl.ds(j, 1), :] = hkv_ref[pl.ds(ki_s[j], 1), :]
    qv_col = _col_f32(qv_s, KH)
    kv_col = _col_f32(kv_s, KH)
    qs = qs_ref[...]
    ks = ks_ref[...] * jax.nn.sigmoid(kv_col)
    qp = _dot(qs, wq_ref[...])
    kp = _dot(ks, wk_ref[...])
    vp = _dot(ks, wv_ref[...])
    if causal:
        mask = _row_i32(ki_s, KH) <= _col_i32(qi_s, KH)
    heads = []
    for h in range(NH):
        sl = slice(h * DH, (h + 1) * DH)
        lh = _dot_t(qp[:, sl], kp[:, sl]) * 0.125
        if causal:
            lh = jnp.where(mask, lh, NEG)
        heads.append(_dot(_softmax(lh), vp[:, sl]))
    o = jnp.concatenate(heads, axis=1)
    hout = _dot(o, wo_ref[...]) * jax.nn.sigmoid(qv_col)
    o_ref[...] = x_ref[...]
    for j in range(KH):
        idx = qi_s[j]
        o_ref[pl.ds(idx, 1), :] = o_ref[pl.ds(idx, 1), :] + hout[j:j + 1, :]


def _heavy(x, hq, hkv, rq, rkv, wq, wk, wv, wo, causal):
    return pl.pallas_call(
        functools.partial(_heavy_body, causal),
        out_shape=jax.ShapeDtypeStruct((T, D), F32),
        scratch_shapes=[
            pltpu.SMEM((KH,), jnp.int32),
            pltpu.SMEM((KH,), F32),
            pltpu.SMEM((KH,), jnp.int32),
            pltpu.SMEM((KH,), F32),
            pltpu.VMEM((KH, D), F32),
            pltpu.VMEM((KH, D), F32),
        ],
    )(x, hq, hkv, rq.reshape(1, D), rkv.reshape(1, D), wq, wk, wv, wo)


def _ffa_body(x_ref, w1_ref, w2_ref, y_ref, h2_ref):
    h2 = _ln2(x_ref[...])
    h2_ref[...] = h2
    y_ref[...] = x_ref[...] + _dot(jax.nn.gelu(_dot(h2, w1_ref[...])),
                                   w2_ref[...])


def _ffa(x, w1, w2):
    return pl.pallas_call(
        _ffa_body,
        grid=(8,),
        in_specs=[
            pl.BlockSpec((T // 8, D), lambda i: (i, 0)),
            pl.BlockSpec((D, D // 2), lambda i: (0, 0)),
            pl.BlockSpec((D // 2, D), lambda i: (0, 0)),
        ],
        out_specs=[
            pl.BlockSpec((T // 8, D), lambda i: (i, 0)),
            pl.BlockSpec((T // 8, D), lambda i: (i, 0)),
        ],
        out_shape=[
            jax.ShapeDtypeStruct((T, D), F32),
            jax.ShapeDtypeStruct((T, D), F32),
        ],
    )(x, w1, w2)


def _ffb_body(y_ref, h2_ref, rf_ref, w1_ref, w2_ref, o_ref, fi_s, fv_s,
              sel_ref):
    _topk_smem(h2_ref, rf_ref, fi_s, fv_s)
    for j in range(KH):
        sel_ref[pl.ds(j, 1), :] = h2_ref[pl.ds(fi_s[j], 1), :]
    fv_col = _col_f32(fv_s, KH)
    mid = jax.nn.gelu(_dot(sel_ref[...], w1_ref[...]))
    hff = _dot(mid, w2_ref[...]) * jax.nn.sigmoid(fv_col)
    o_ref[...] = y_ref[...]
    for j in range(KH):
        idx = fi_s[j]
        o_ref[pl.ds(idx, 1), :] = o_ref[pl.ds(idx, 1), :] + hff[j:j + 1, :]


def _ffb(y, h2, rf, w1, w2):
    return pl.pallas_call(
        _ffb_body,
        out_shape=jax.ShapeDtypeStruct((T, D), F32),
        scratch_shapes=[
            pltpu.SMEM((KH,), jnp.int32),
            pltpu.SMEM((KH,), F32),
            pltpu.VMEM((KH, D), F32),
        ],
    )(y, h2, rf.reshape(1, D), w1, w2)


def _lm_body(x_ref, w_ref, b_ref, o_ref):
    o_ref[...] = _dot(x_ref[...], w_ref[...]) + b_ref[...]


def _lm_head(x, w, b):
    return pl.pallas_call(
        _lm_body,
        grid=(pl.cdiv(VOCAB, VB),),
        in_specs=[
            pl.BlockSpec((T, D), lambda i: (0, 0)),
            pl.BlockSpec((D, VB), lambda i: (0, i)),
            pl.BlockSpec((1, VB), lambda i: (0, i)),
        ],
        out_specs=pl.BlockSpec((T, VB), lambda i: (0, i)),
        out_shape=jax.ShapeDtypeStruct((T, VOCAB), F32),
    )(x, w, b.reshape(1, VOCAB))


def _enc_block(x, p):
    h = _ln(x)
    x = _light(x, h, h, p['lq'], p['lk'], p['lv'], p['lo'], False)
    x = _heavy(x, h, h, p['rq'], p['rkv'], p['hq'], p['hk'], p['hv'],
               p['ho'], False)
    y, h2 = _ffa(x, p['lf1'], p['lf2'])
    return _ffb(y, h2, p['rf'], p['hf1'], p['hf2'])


def _dec_block(x, e_ln, p):
    h = _ln(x)
    x = _light(x, h, h, p['lq'], p['lk'], p['lv'], p['lo'], True)
    x = _heavy(x, h, h, p['rq'], p['rkv'], p['hq'], p['hk'], p['hv'],
               p['ho'], True)
    h2 = _ln(x)
    x = _light(x, h2, e_ln, p['clq'], p['clk'], p['clv'], p['clo'], False)
    x = _heavy(x, h2, e_ln, p['crq'], p['crkv'], p['cq'], p['ck'], p['cv'],
               p['co'], False)
    y, hh = _ffa(x, p['lf1'], p['lf2'])
    return _ffb(y, hh, p['rf'], p['hf1'], p['hf2'])


def kernel(params, input_ids, decoder_input_ids):
    p = params
    x = _embed(p['enc_embed'], input_ids.reshape(-1))
    for lp in p['enc_layers']:
        x = _enc_block(x, lp)
    e_ln = _ln(x)
    y = _embed(p['dec_embed'], decoder_input_ids.reshape(-1))
    for lp in p['dec_layers']:
        y = _dec_block(y, e_ln, lp)
    logits = _lm_head(y, p['lm_w'], p['lm_b'])
    return logits.reshape(1, T, VOCAB)
